# Initial kernel scaffold; baseline (speedup 1.0000x reference)
import jax
import jax.numpy as jnp
from jax.experimental import pallas as pl


def kernel(embs, lengths, padded_seq_len, W_map, b_map, W_enc, b_enc):
    raise NotImplementedError("write your pallas kernel here")


# speedup vs baseline: 1.6197x; 1.6197x over previous
import functools

import jax
import jax.numpy as jnp
from jax.experimental import pallas as pl
from jax.experimental.pallas import tpu as pltpu


def _combine_body(wm_ref, we_ref, bm_ref, be_ref, wc_ref, bc_ref):
    wc_ref[...] = jnp.dot(wm_ref[...], we_ref[...],
                          preferred_element_type=jnp.float32)
    bc_ref[...] = jnp.dot(bm_ref[...], we_ref[...],
                          preferred_element_type=jnp.float32) + be_ref[...]


def _encode_body(lens_ref, psl_ref, x_ref, wc_ref, bc_ref, o_ref, *, tb, nj):
    b = pl.program_id(0)
    j = pl.program_id(1)
    length = lens_ref[b]
    eff = jnp.minimum(length, psl_ref[0])

    @pl.when(j == 0)
    def _init():
        o_ref[...] = jnp.zeros_like(o_ref)

    start = j * tb

    @pl.when(start < eff)
    def _accumulate():
        h = jnp.dot(x_ref[...], wc_ref[...],
                    preferred_element_type=jnp.float32)
        h = jnp.maximum(h + bc_ref[...], 0.0)
        pos = start + jax.lax.broadcasted_iota(jnp.int32, (tb, 1), 0)
        mask = (pos < eff).astype(jnp.float32)
        o_ref[...] += jnp.sum(h * mask, axis=0).reshape(o_ref.shape)

    @pl.when(j == nj - 1)
    def _finalize():
        denom = jnp.maximum(length, 1).astype(jnp.float32)
        o_ref[...] = o_ref[...] / denom


def kernel(embs, lengths, padded_seq_len, W_map, b_map, W_enc, b_enc):
    n_tok, prev = embs.shape
    hid = W_enc.shape[1]
    nb = lengths.shape[0]
    max_len = n_tok // nb

    wc, bc = pl.pallas_call(
        _combine_body,
        out_shape=[
            jax.ShapeDtypeStruct((prev, hid), jnp.float32),
            jax.ShapeDtypeStruct((1, hid), jnp.float32),
        ],
    )(W_map, W_enc, b_map.reshape(1, hid), b_enc.reshape(1, hid))

    tb = 256
    nj = max_len // tb
    lens = lengths.astype(jnp.int32)
    psl = jnp.asarray(padded_seq_len, jnp.int32).reshape(1)

    def x_map(b, j, lens_ref, psl_ref):
        eff = jnp.minimum(jnp.minimum(lens_ref[b], psl_ref[0]), max_len)
        nj_used = jnp.maximum((eff + tb - 1) // tb, 1)
        return (b * nj + jnp.minimum(j, nj_used - 1), 0)

    out = pl.pallas_call(
        functools.partial(_encode_body, tb=tb, nj=nj),
        grid_spec=pltpu.PrefetchScalarGridSpec(
            num_scalar_prefetch=2,
            grid=(nb, nj),
            in_specs=[
                pl.BlockSpec((tb, prev), x_map),
                pl.BlockSpec((prev, hid), lambda b, j, *_: (0, 0)),
                pl.BlockSpec((1, hid), lambda b, j, *_: (0, 0)),
            ],
            out_specs=pl.BlockSpec((1, 1, hid), lambda b, j, *_: (b, 0, 0)),
        ),
        out_shape=jax.ShapeDtypeStruct((nb, 1, hid), jnp.float32),
    )(lens, psl, embs, wc, bc)
    return out.reshape(nb, hid)


# speedup vs baseline: 1.6363x; 1.0103x over previous
import functools

import jax
import jax.numpy as jnp
from jax.experimental import pallas as pl
from jax.experimental.pallas import tpu as pltpu


def _combine_body(wm_ref, we_ref, bm_ref, be_ref, wc_ref, bc_ref):
    wc_ref[...] = jnp.dot(wm_ref[...], we_ref[...],
                          preferred_element_type=jnp.float32
                          ).astype(jnp.bfloat16)
    bc_ref[...] = jnp.dot(bm_ref[...], we_ref[...],
                          preferred_element_type=jnp.float32) + be_ref[...]


def _encode_body(lens_ref, psl_ref, x_ref, wc_ref, bc_ref, o_ref, *, tb, nj):
    b = pl.program_id(0)
    j = pl.program_id(1)
    length = lens_ref[b]
    eff = jnp.minimum(length, psl_ref[0])

    @pl.when(j == 0)
    def _init():
        o_ref[...] = jnp.zeros_like(o_ref)

    start = j * tb

    @pl.when(start < eff)
    def _accumulate():
        h = jnp.dot(x_ref[...].astype(jnp.bfloat16), wc_ref[...],
                    preferred_element_type=jnp.float32)
        h = jnp.maximum(h + bc_ref[...], 0.0)
        pos = start + jax.lax.broadcasted_iota(jnp.int32, (tb, 1), 0)
        mask = (pos < eff).astype(jnp.float32)
        o_ref[...] += jnp.sum(h * mask, axis=0).reshape(o_ref.shape)

    @pl.when(j == nj - 1)
    def _finalize():
        denom = jnp.maximum(length, 1).astype(jnp.float32)
        o_ref[...] = o_ref[...] / denom


def kernel(embs, lengths, padded_seq_len, W_map, b_map, W_enc, b_enc):
    n_tok, prev = embs.shape
    hid = W_enc.shape[1]
    nb = lengths.shape[0]
    max_len = n_tok // nb

    wc, bc = pl.pallas_call(
        _combine_body,
        out_shape=[
            jax.ShapeDtypeStruct((prev, hid), jnp.bfloat16),
            jax.ShapeDtypeStruct((1, hid), jnp.float32),
        ],
    )(W_map, W_enc, b_map.reshape(1, hid), b_enc.reshape(1, hid))

    tb = 256
    nj = max_len // tb
    lens = lengths.astype(jnp.int32)
    psl = jnp.asarray(padded_seq_len, jnp.int32).reshape(1)

    def x_map(b, j, lens_ref, psl_ref):
        eff = jnp.minimum(jnp.minimum(lens_ref[b], psl_ref[0]), max_len)
        nj_used = jnp.maximum((eff + tb - 1) // tb, 1)
        return (b * nj + jnp.minimum(j, nj_used - 1), 0)

    out = pl.pallas_call(
        functools.partial(_encode_body, tb=tb, nj=nj),
        grid_spec=pltpu.PrefetchScalarGridSpec(
            num_scalar_prefetch=2,
            grid=(nb, nj),
            in_specs=[
                pl.BlockSpec((tb, prev), x_map),
                pl.BlockSpec((prev, hid), lambda b, j, *_: (0, 0)),
                pl.BlockSpec((1, hid), lambda b, j, *_: (0, 0)),
            ],
            out_specs=pl.BlockSpec((1, 1, hid), lambda b, j, *_: (b, 0, 0)),
        ),
        out_shape=jax.ShapeDtypeStruct((nb, 1, hid), jnp.float32),
    )(lens, psl, embs, wc, bc)
    return out.reshape(nb, hid)
